# baseline (speedup 1.0000x reference)
import jax
import jax.numpy as jnp
from jax.experimental import pallas as pl
from jax.experimental.pallas import tpu as pltpu

_TM = 1024
_VMEM_LIMIT_BYTES = 48 * 1024 * 1024


def _body(x_ref, b_ref, w_hbm, o_ref, wbuf, sem_w):
    i = pl.program_id(0)
    F = wbuf.shape[0]
    k2 = F // 2

    def cp_w(half):
        return pltpu.make_async_copy(
            w_hbm.at[pl.ds(half * k2, k2), :],
            wbuf.at[pl.ds(half * k2, k2), :],
            sem_w.at[half],
        )

    @pl.when(i == 0)
    def _():
        cp_w(0).start()
        cp_w(1).start()
        cp_w(0).wait()
        o_ref[...] = (
            jnp.dot(x_ref[:, :k2], wbuf[pl.ds(0, k2), :],
                    preferred_element_type=jnp.float32)
            + b_ref[...]
        )
        cp_w(1).wait()
        o_ref[...] += jnp.dot(x_ref[:, k2:], wbuf[pl.ds(k2, k2), :],
                              preferred_element_type=jnp.float32)

    @pl.when(i > 0)
    def _():
        o_ref[...] = (
            jnp.dot(x_ref[...], wbuf[...], preferred_element_type=jnp.float32)
            + b_ref[...]
        )


def kernel(x, w_packed, b_packed):
    B, F = x.shape
    C = w_packed.shape[1]

    tm = _TM if B % _TM == 0 else B
    grid = (B // tm,)

    cost = pl.CostEstimate(
        flops=2 * B * C * F,
        transcendentals=0,
        bytes_accessed=4 * (B * F + F * C + B * C),
    )
    return pl.pallas_call(
        _body,
        out_shape=jax.ShapeDtypeStruct((B, C), jnp.float32),
        grid=grid,
        in_specs=[
            pl.BlockSpec((tm, F), lambda i: (i, 0)),
            pl.BlockSpec((1, C), lambda i: (0, 0)),
            pl.BlockSpec(memory_space=pl.ANY),
        ],
        out_specs=pl.BlockSpec((tm, C), lambda i: (i, 0)),
        scratch_shapes=[
            pltpu.VMEM((F, C), jnp.float32),
            pltpu.SemaphoreType.DMA((2,)),
        ],
        compiler_params=pltpu.CompilerParams(
            dimension_semantics=("arbitrary",),
            vmem_limit_bytes=_VMEM_LIMIT_BYTES,
        ),
        cost_estimate=cost,
    )(x, b_packed, w_packed)


# speedup vs baseline: 1.0558x; 1.0558x over previous
import jax
import jax.numpy as jnp
from jax.experimental import pallas as pl
from jax.experimental.pallas import tpu as pltpu

_TM = 1024
_VMEM_LIMIT_BYTES = 48 * 1024 * 1024


def _body(x_ref, w_ref, b_ref, o_ref):
    o_ref[...] = (
        jnp.dot(x_ref[...], w_ref[...], preferred_element_type=jnp.float32)
        + b_ref[...]
    )


def kernel(x, w_packed, b_packed):
    B, F = x.shape
    C = w_packed.shape[1]

    tm = _TM if B % _TM == 0 else B
    grid = (B // tm,)

    cost = pl.CostEstimate(
        flops=2 * B * C * F,
        transcendentals=0,
        bytes_accessed=4 * (B * F + F * C + B * C),
    )
    return pl.pallas_call(
        _body,
        out_shape=jax.ShapeDtypeStruct((B, C), jnp.float32),
        grid=grid,
        in_specs=[
            pl.BlockSpec((tm, F), lambda i: (i, 0)),
            pl.BlockSpec((F, C), lambda i: (0, 0)),
            pl.BlockSpec((1, C), lambda i: (0, 0)),
        ],
        out_specs=pl.BlockSpec((tm, C), lambda i: (i, 0)),
        compiler_params=pltpu.CompilerParams(
            dimension_semantics=("parallel",),
            vmem_limit_bytes=_VMEM_LIMIT_BYTES,
        ),
        cost_estimate=cost,
    )(x, w_packed, b_packed)
